# baseline (speedup 1.0000x reference)
import jax
import jax.numpy as jnp
from jax.experimental import pallas as pl
from jax.experimental.pallas import tpu as pltpu
import functools

_B, _N, _C, _P, _TOPK = 4, 20000, 80, 32, 300
_D = 2 * _P + 4
_QB = 1024
_NBLK = 20
_T = 160
_NCOPY = _B * _NBLK
_UNROLL = 4


def _body(lg, c2, b2, ot, lb_o, bo_o, cd_o, sc_o, f3, l1, qs, ls, ss):
    g = pl.program_id(0)

    @pl.when(g < _NCOPY)
    def _copy():
        b = g // _NBLK
        a = g - b * _NBLK
        x = lg[...]
        qabs = (jax.lax.broadcasted_iota(jnp.int32, (1, _QB, _C), 1)
                + a * _QB)
        x = jnp.where(qabs < _N, x, -jnp.inf)
        for j in range(_QB // 128):
            xj = x[:, j * 128:(j + 1) * 128, :]
            t = a * (_QB // 128) + j
            f3[pl.ds(b, 1), pl.ds(t, 1), :, :] = xj[None]
            l1[pl.ds(b, 1), pl.ds(t, 1), :] = jnp.max(xj, axis=2)[None]

    @pl.when(g == _NCOPY - 1)
    def _select():
        riota = (jax.lax.broadcasted_iota(jnp.int32, (_T, 128), 0) * 128
                 + jax.lax.broadcasted_iota(jnp.int32, (_T, 128), 1))
        liota = jax.lax.broadcasted_iota(jnp.int32, (1, 1, _C), 2)
        siota = jax.lax.broadcasted_iota(jnp.int32, (1, 128), 1)
        big = jnp.int32(1 << 30)
        neg_inf = jnp.float32(-jnp.inf)

        def step(i, carry):
            for b in range(_B):
                l1v = l1[b]
                m = jnp.max(l1v)
                q = jnp.min(jnp.where(l1v == m, riota, big))
                t = q // 128
                s = q - t * 128
                row = f3[b, pl.ds(t, 1), pl.ds(s, 1), :]
                lane = jnp.min(jnp.where(row == m, liota, big))
                newrow = jnp.where(liota == lane, neg_inf, row)
                f3[b, pl.ds(t, 1), pl.ds(s, 1), :] = newrow
                l1row = l1[b, pl.ds(t, 1), :]
                l1[b, pl.ds(t, 1), :] = jnp.where(
                    siota == s, jnp.max(newrow, axis=2), l1row)
                qs[b, pl.ds(i, 1), :] = jnp.reshape(q, (1, 1))
                ls[b, pl.ds(i, 1), :] = jnp.reshape(lane, (1, 1))
                ss[b, pl.ds(i, 1), :] = jnp.reshape(jax.nn.sigmoid(m),
                                                    (1, 1))
            return carry

        jax.lax.fori_loop(0, _TOPK, step, 0)

    @pl.when(g >= _NCOPY)
    def _gather():
        b = g - _NCOPY
        o0 = ot[0, 0, 0]
        o1 = ot[0, 0, 1]
        cio = jax.lax.broadcasted_iota(jnp.int32, (1, 1, 2 * _P), 2)
        svec = jnp.where(cio % 2 == 0, o0, o1)

        def gstep(i, carry):
            for k in range(_UNROLL):
                i4 = i * _UNROLL + k
                q = qs[pl.ds(b, 1), pl.ds(i4, 1), :][0, 0, 0]
                rowc = c2[:, pl.ds(q // 2, 1), :]
                half = jnp.where(q % 2 == 0,
                                 rowc[:, :, 0:64], rowc[:, :, 64:128])
                cd_o[:, pl.ds(i4, 1), :] = half * svec
                rowb = b2[:, pl.ds(q // 32, 1), :]
                jm = q - (q // 32) * 32
                v64 = jnp.where(jm < 16, rowb[:, :, 0:64], rowb[:, :, 64:128])
                v32 = jnp.where(jm % 16 < 8, v64[:, :, 0:32], v64[:, :, 32:64])
                v16 = jnp.where(jm % 8 < 4, v32[:, :, 0:16], v32[:, :, 16:32])
                v8 = jnp.where(jm % 4 < 2, v16[:, :, 0:8], v16[:, :, 8:16])
                box4 = jnp.where(jm % 2 < 1, v8[:, :, 0:4], v8[:, :, 4:8])
                cx = box4[0, 0, 0]
                cy = box4[0, 0, 1]
                w = box4[0, 0, 2]
                h = box4[0, 0, 3]
                x0 = (cx - 0.5 * w) * o0
                y0 = (cy - 0.5 * h) * o1
                x1 = (cx + 0.5 * w) * o0
                y1 = (cy + 0.5 * h) * o1
                biota = jax.lax.broadcasted_iota(jnp.int32, (1, 1, 4), 2)
                bo_o[:, pl.ds(i4, 1), :] = jnp.where(
                    biota == 0, x0, jnp.where(biota == 1, y0,
                                              jnp.where(biota == 2, x1, y1)))
                sc_o[:, pl.ds(i4, 1), :] = ss[pl.ds(b, 1), pl.ds(i4, 1), :]
                lb_o[:, pl.ds(i4, 1), :] = ls[pl.ds(b, 1), pl.ds(i4, 1), :]
            return carry

        jax.lax.fori_loop(0, _TOPK // _UNROLL, gstep, 0)


@functools.partial(jax.jit, static_argnames=("interpret",))
def _run(pred_logits, pred_coords, pred_boxes, orig_target_sizes, interpret=False):
    coords2 = pred_coords.reshape(_B, _N // 2, 128)
    boxes2 = pred_boxes.reshape(_B, _N // 32, 128)
    ots = orig_target_sizes.reshape(_B, 1, 2)

    def _lg_map(g):
        return (jnp.minimum(g // _NBLK, _B - 1),
                jnp.where(g < _NCOPY, g % _NBLK, _NBLK - 1), 0)

    def _out_map(g):
        return (jnp.clip(g - _NCOPY, 0, _B - 1), 0, 0)

    grid = (_NCOPY + _B,)
    lb, bo, cd, sc = pl.pallas_call(
        _body,
        grid=grid,
        in_specs=[
            pl.BlockSpec((1, _QB, _C), _lg_map),
            pl.BlockSpec((1, _N // 2, 128), _out_map),
            pl.BlockSpec((1, _N // 32, 128), _out_map),
            pl.BlockSpec((1, 1, 2), _out_map),
        ],
        out_specs=[
            pl.BlockSpec((1, _TOPK, 1), _out_map),
            pl.BlockSpec((1, _TOPK, 4), _out_map),
            pl.BlockSpec((1, _TOPK, 2 * _P), _out_map),
            pl.BlockSpec((1, _TOPK, 1), _out_map),
        ],
        out_shape=[
            jax.ShapeDtypeStruct((_B, _TOPK, 1), jnp.int32),
            jax.ShapeDtypeStruct((_B, _TOPK, 4), jnp.float32),
            jax.ShapeDtypeStruct((_B, _TOPK, 2 * _P), jnp.float32),
            jax.ShapeDtypeStruct((_B, _TOPK, 1), jnp.float32),
        ],
        scratch_shapes=[
            pltpu.VMEM((_B, _T, 128, _C), jnp.float32),
            pltpu.VMEM((_B, _T, 128), jnp.float32),
            pltpu.VMEM((_B, _TOPK, 1), jnp.int32),
            pltpu.VMEM((_B, _TOPK, 1), jnp.int32),
            pltpu.VMEM((_B, _TOPK, 1), jnp.float32),
        ],
        interpret=interpret,
    )(pred_logits, coords2, boxes2, ots)

    labels = lb.reshape(_B, _TOPK)
    boxes_sel = bo
    coords_sel = cd.reshape(_B, _TOPK, _P, 2)
    top_scores = sc.reshape(_B, _TOPK)
    return labels, boxes_sel, coords_sel, top_scores


def kernel(pred_logits, pred_coords, pred_boxes, orig_target_sizes, input_sizes):
    return _run(pred_logits, pred_coords, pred_boxes, orig_target_sizes)


# speedup vs baseline: 1.0061x; 1.0061x over previous
import jax
import jax.numpy as jnp
from jax.experimental import pallas as pl
from jax.experimental.pallas import tpu as pltpu
import functools

_B, _N, _C, _P, _TOPK = 4, 20000, 80, 32, 300
_QB = 2048
_NBLK = 10
_T = 160
_UNROLL = 4


def _body(lg, c2, b2, ot, lb_o, bo_o, cd_o, sc_o, f3, l1, qs, ls, ss):
    a = pl.program_id(1)

    x = lg[...]
    qabs = (jax.lax.broadcasted_iota(jnp.int32, (1, _QB, _C), 1)
            + a * _QB)
    x = jnp.where(qabs < _N, x, -jnp.inf)
    for j in range(_QB // 128):
        xj = x[:, j * 128:(j + 1) * 128, :]
        t = a * (_QB // 128) + j
        f3[pl.ds(t, 1), :, :] = xj
        l1[pl.ds(t, 1), :] = jnp.max(xj, axis=2)

    @pl.when(a == _NBLK - 1)
    def _select_and_gather():
        riota = (jax.lax.broadcasted_iota(jnp.int32, (_T, 128), 0) * 128
                 + jax.lax.broadcasted_iota(jnp.int32, (_T, 128), 1))
        liota = jax.lax.broadcasted_iota(jnp.int32, (1, 1, _C), 2)
        siota = jax.lax.broadcasted_iota(jnp.int32, (1, 128), 1)
        big = jnp.int32(1 << 30)
        neg_inf = jnp.float32(-jnp.inf)

        def step(i, carry):
            l1v = l1[...]
            m = jnp.max(l1v)
            q = jnp.min(jnp.where(l1v == m, riota, big))
            t = q // 128
            s = q - t * 128
            row = f3[pl.ds(t, 1), pl.ds(s, 1), :]
            lane = jnp.min(jnp.where(row == m, liota, big))
            newrow = jnp.where(liota == lane, neg_inf, row)
            f3[pl.ds(t, 1), pl.ds(s, 1), :] = newrow
            l1row = l1[pl.ds(t, 1), :]
            l1[pl.ds(t, 1), :] = jnp.where(siota == s,
                                           jnp.max(newrow, axis=2), l1row)
            qs[pl.ds(i, 1), :] = jnp.reshape(q, (1, 1))
            ls[pl.ds(i, 1), :] = jnp.reshape(lane, (1, 1))
            ss[pl.ds(i, 1), :] = jnp.reshape(jax.nn.sigmoid(m), (1, 1))
            return carry

        jax.lax.fori_loop(0, _TOPK, step, 0)

        o0 = ot[0, 0, 0]
        o1 = ot[0, 0, 1]
        cio = jax.lax.broadcasted_iota(jnp.int32, (1, 1, 2 * _P), 2)
        svec = jnp.where(cio % 2 == 0, o0, o1)
        biota = jax.lax.broadcasted_iota(jnp.int32, (1, 1, 4), 2)

        def gstep(i, carry):
            for k in range(_UNROLL):
                i4 = i * _UNROLL + k
                q = qs[pl.ds(i4, 1), :][0, 0]
                rowc = c2[:, pl.ds(q // 2, 1), :]
                half = jnp.where(q % 2 == 0,
                                 rowc[:, :, 0:64], rowc[:, :, 64:128])
                cd_o[:, pl.ds(i4, 1), :] = half * svec
                rowb = b2[:, pl.ds(q // 32, 1), :]
                jm = q - (q // 32) * 32
                v64 = jnp.where(jm < 16, rowb[:, :, 0:64], rowb[:, :, 64:128])
                v32 = jnp.where(jm % 16 < 8, v64[:, :, 0:32], v64[:, :, 32:64])
                v16 = jnp.where(jm % 8 < 4, v32[:, :, 0:16], v32[:, :, 16:32])
                v8 = jnp.where(jm % 4 < 2, v16[:, :, 0:8], v16[:, :, 8:16])
                box4 = jnp.where(jm % 2 < 1, v8[:, :, 0:4], v8[:, :, 4:8])
                cx = box4[0, 0, 0]
                cy = box4[0, 0, 1]
                w = box4[0, 0, 2]
                h = box4[0, 0, 3]
                x0 = (cx - 0.5 * w) * o0
                y0 = (cy - 0.5 * h) * o1
                x1 = (cx + 0.5 * w) * o0
                y1 = (cy + 0.5 * h) * o1
                bo_o[:, pl.ds(i4, 1), :] = jnp.where(
                    biota == 0, x0, jnp.where(biota == 1, y0,
                                              jnp.where(biota == 2, x1, y1)))
                sc_o[:, pl.ds(i4, 1), :] = ss[pl.ds(i4, 1), :][None]
                lb_o[:, pl.ds(i4, 1), :] = ls[pl.ds(i4, 1), :][None]
            return carry

        jax.lax.fori_loop(0, _TOPK // _UNROLL, gstep, 0)


@functools.partial(jax.jit, static_argnames=("interpret",))
def _run(pred_logits, pred_coords, pred_boxes, orig_target_sizes, interpret=False):
    coords2 = pred_coords.reshape(_B, _N // 2, 128)
    boxes2 = pred_boxes.reshape(_B, _N // 32, 128)
    ots = orig_target_sizes.reshape(_B, 1, 2)

    grid = (_B, _NBLK)
    lb, bo, cd, sc = pl.pallas_call(
        _body,
        grid=grid,
        in_specs=[
            pl.BlockSpec((1, _QB, _C), lambda b, a: (b, a, 0)),
            pl.BlockSpec((1, _N // 2, 128), lambda b, a: (b, 0, 0)),
            pl.BlockSpec((1, _N // 32, 128), lambda b, a: (b, 0, 0)),
            pl.BlockSpec((1, 1, 2), lambda b, a: (b, 0, 0)),
        ],
        out_specs=[
            pl.BlockSpec((1, _TOPK, 1), lambda b, a: (b, 0, 0)),
            pl.BlockSpec((1, _TOPK, 4), lambda b, a: (b, 0, 0)),
            pl.BlockSpec((1, _TOPK, 2 * _P), lambda b, a: (b, 0, 0)),
            pl.BlockSpec((1, _TOPK, 1), lambda b, a: (b, 0, 0)),
        ],
        out_shape=[
            jax.ShapeDtypeStruct((_B, _TOPK, 1), jnp.int32),
            jax.ShapeDtypeStruct((_B, _TOPK, 4), jnp.float32),
            jax.ShapeDtypeStruct((_B, _TOPK, 2 * _P), jnp.float32),
            jax.ShapeDtypeStruct((_B, _TOPK, 1), jnp.float32),
        ],
        scratch_shapes=[
            pltpu.VMEM((_T, 128, _C), jnp.float32),
            pltpu.VMEM((_T, 128), jnp.float32),
            pltpu.VMEM((_TOPK, 1), jnp.int32),
            pltpu.VMEM((_TOPK, 1), jnp.int32),
            pltpu.VMEM((_TOPK, 1), jnp.float32),
        ],
        interpret=interpret,
    )(pred_logits, coords2, boxes2, ots)

    labels = lb.reshape(_B, _TOPK)
    boxes_sel = bo
    coords_sel = cd.reshape(_B, _TOPK, _P, 2)
    top_scores = sc.reshape(_B, _TOPK)
    return labels, boxes_sel, coords_sel, top_scores


def kernel(pred_logits, pred_coords, pred_boxes, orig_target_sizes, input_sizes):
    return _run(pred_logits, pred_coords, pred_boxes, orig_target_sizes)


# speedup vs baseline: 1.1148x; 1.1080x over previous
import jax
import jax.numpy as jnp
from jax.experimental import pallas as pl
from jax.experimental.pallas import tpu as pltpu
import functools

_B, _N, _C, _P, _TOPK = 4, 20000, 80, 32, 300
_QB = 2048
_NBLK = 10
_T = 160
_UNROLL = 4


def _body(lg, c2, b2, ot, lb_o, bo_o, cd_o, sc_o, f3, l1, qs, ls, ss):
    a = pl.program_id(1)

    x = lg[...]
    qabs = (jax.lax.broadcasted_iota(jnp.int32, (1, _QB, _C), 1)
            + a * _QB)
    x = jnp.where(qabs < _N, x, -jnp.inf)
    for j in range(_QB // 128):
        xj = x[:, j * 128:(j + 1) * 128, :]
        t = a * (_QB // 128) + j
        f3[pl.ds(t, 1), :, :] = xj
        l1[pl.ds(t, 1), :] = jnp.max(xj, axis=2)

    @pl.when(a == _NBLK - 1)
    def _select_and_gather():
        riota = (jax.lax.broadcasted_iota(jnp.int32, (_T, 128), 0) * 128
                 + jax.lax.broadcasted_iota(jnp.int32, (_T, 128), 1))
        liota = jax.lax.broadcasted_iota(jnp.int32, (1, 1, _C), 2)
        siota = jax.lax.broadcasted_iota(jnp.int32, (1, 128), 1)
        big = jnp.int32(1 << 30)
        neg_inf = jnp.float32(-jnp.inf)

        def step(i, carry):
            l1v = l1[...]
            m = jnp.max(l1v)
            q = jnp.min(jnp.where(l1v == m, riota, big))
            t = q // 128
            s = q - t * 128
            row = f3[pl.ds(t, 1), pl.ds(s, 1), :]
            lane = jnp.min(jnp.where(row == m, liota, big))
            newrow = jnp.where(liota == lane, neg_inf, row)
            f3[pl.ds(t, 1), pl.ds(s, 1), :] = newrow
            l1row = l1[pl.ds(t, 1), :]
            l1[pl.ds(t, 1), :] = jnp.where(siota == s,
                                           jnp.max(newrow, axis=2), l1row)
            qs[pl.ds(i, 1), :] = jnp.reshape(q, (1, 1))
            ls[pl.ds(i, 1), :] = jnp.reshape(lane, (1, 1))
            ss[pl.ds(i, 1), :] = jnp.reshape(jax.nn.sigmoid(m), (1, 1))
            return carry

        jax.lax.fori_loop(0, _TOPK, step, 0)

        o0 = ot[0, 0, 0]
        o1 = ot[0, 0, 1]
        cio = jax.lax.broadcasted_iota(jnp.int32, (1, 1, 2 * _P), 2)
        svec = jnp.where(cio % 2 == 0, o0, o1)
        b4io = jax.lax.broadcasted_iota(jnp.int32, (1, 1, 4), 2)
        ovec4 = jnp.where(b4io % 2 == 0, o0, o1)

        def _blend(lo, hi, pbit):
            return lo + (hi - lo) * pbit

        def gstep(i, carry):
            for k in range(_UNROLL):
                i4 = i * _UNROLL + k
                q = qs[pl.ds(i4, 1), :][0, 0]
                rowc = c2[:, pl.ds(q // 2, 1), :]
                p0 = (q % 2).astype(jnp.float32)
                half = _blend(rowc[:, :, 0:64], rowc[:, :, 64:128], p0)
                cd_o[:, pl.ds(i4, 1), :] = half * svec
                rowb = b2[:, pl.ds(q // 32, 1), :]
                jm = q - (q // 32) * 32
                p4 = (jm // 16).astype(jnp.float32)
                p3 = (jm // 8 % 2).astype(jnp.float32)
                p2 = (jm // 4 % 2).astype(jnp.float32)
                p1 = (jm // 2 % 2).astype(jnp.float32)
                pp0 = (jm % 2).astype(jnp.float32)
                v64 = _blend(rowb[:, :, 0:64], rowb[:, :, 64:128], p4)
                v32 = _blend(v64[:, :, 0:32], v64[:, :, 32:64], p3)
                v16 = _blend(v32[:, :, 0:16], v32[:, :, 16:32], p2)
                v8 = _blend(v16[:, :, 0:8], v16[:, :, 8:16], p1)
                box4 = _blend(v8[:, :, 0:4], v8[:, :, 4:8], pp0)
                ab = box4[:, :, 0:2]
                wh = box4[:, :, 2:4]
                xy = jnp.concatenate([ab - 0.5 * wh, ab + 0.5 * wh], axis=2)
                bo_o[:, pl.ds(i4, 1), :] = xy * ovec4
                sc_o[:, pl.ds(i4, 1), :] = ss[pl.ds(i4, 1), :][None]
                lb_o[:, pl.ds(i4, 1), :] = ls[pl.ds(i4, 1), :][None]
            return carry

        jax.lax.fori_loop(0, _TOPK // _UNROLL, gstep, 0)


@functools.partial(jax.jit, static_argnames=("interpret",))
def _run(pred_logits, pred_coords, pred_boxes, orig_target_sizes, interpret=False):
    coords2 = pred_coords.reshape(_B, _N // 2, 128)
    boxes2 = pred_boxes.reshape(_B, _N // 32, 128)
    ots = orig_target_sizes.reshape(_B, 1, 2)

    grid = (_B, _NBLK)
    lb, bo, cd, sc = pl.pallas_call(
        _body,
        grid=grid,
        in_specs=[
            pl.BlockSpec((1, _QB, _C), lambda b, a: (b, a, 0)),
            pl.BlockSpec((1, _N // 2, 128), lambda b, a: (b, 0, 0)),
            pl.BlockSpec((1, _N // 32, 128), lambda b, a: (b, 0, 0)),
            pl.BlockSpec((1, 1, 2), lambda b, a: (b, 0, 0)),
        ],
        out_specs=[
            pl.BlockSpec((1, _TOPK, 1), lambda b, a: (b, 0, 0)),
            pl.BlockSpec((1, _TOPK, 4), lambda b, a: (b, 0, 0)),
            pl.BlockSpec((1, _TOPK, 2 * _P), lambda b, a: (b, 0, 0)),
            pl.BlockSpec((1, _TOPK, 1), lambda b, a: (b, 0, 0)),
        ],
        out_shape=[
            jax.ShapeDtypeStruct((_B, _TOPK, 1), jnp.int32),
            jax.ShapeDtypeStruct((_B, _TOPK, 4), jnp.float32),
            jax.ShapeDtypeStruct((_B, _TOPK, 2 * _P), jnp.float32),
            jax.ShapeDtypeStruct((_B, _TOPK, 1), jnp.float32),
        ],
        scratch_shapes=[
            pltpu.VMEM((_T, 128, _C), jnp.float32),
            pltpu.VMEM((_T, 128), jnp.float32),
            pltpu.VMEM((_TOPK, 1), jnp.int32),
            pltpu.VMEM((_TOPK, 1), jnp.int32),
            pltpu.VMEM((_TOPK, 1), jnp.float32),
        ],
        interpret=interpret,
    )(pred_logits, coords2, boxes2, ots)

    labels = lb.reshape(_B, _TOPK)
    boxes_sel = bo
    coords_sel = cd.reshape(_B, _TOPK, _P, 2)
    top_scores = sc.reshape(_B, _TOPK)
    return labels, boxes_sel, coords_sel, top_scores


def kernel(pred_logits, pred_coords, pred_boxes, orig_target_sizes, input_sizes):
    return _run(pred_logits, pred_coords, pred_boxes, orig_target_sizes)


# speedup vs baseline: 3.2058x; 2.8758x over previous
import jax
import jax.numpy as jnp
from jax.experimental import pallas as pl
from jax.experimental.pallas import tpu as pltpu
import functools

_B, _N, _C, _P, _TOPK = 4, 20000, 80, 32, 300
_D = 2 * _P + 4
_QB = 2048
_NBLK = 10
_T = 160


def _body(lg, cb, ot, lb_o, bo_o, cd_o, sc_o, f3, l1):
    a = pl.program_id(1)

    x = lg[...]
    qabs = (jax.lax.broadcasted_iota(jnp.int32, (1, _QB, _C), 1)
            + a * _QB)
    x = jnp.where(qabs < _N, x, -jnp.inf)
    for j in range(_QB // 128):
        xj = x[:, j * 128:(j + 1) * 128, :]
        t = a * (_QB // 128) + j
        f3[pl.ds(t, 1), :, :] = xj
        l1[pl.ds(t, 1), :] = jnp.max(xj, axis=2)

    @pl.when(a == _NBLK - 1)
    def _select():
        o0 = ot[0, 0, 0]
        o1 = ot[0, 0, 1]
        cio = jax.lax.broadcasted_iota(jnp.int32, (1, 1, 2 * _P), 2)
        svec = jnp.where(cio % 2 == 0, o0, o1)

        riota = (jax.lax.broadcasted_iota(jnp.int32, (_T, 128), 0) * 128
                 + jax.lax.broadcasted_iota(jnp.int32, (_T, 128), 1))
        liota = jax.lax.broadcasted_iota(jnp.int32, (1, 1, _C), 2)
        siota = jax.lax.broadcasted_iota(jnp.int32, (1, 128), 1)
        biota = jax.lax.broadcasted_iota(jnp.int32, (1, 1, 4), 2)
        big = jnp.int32(1 << 30)
        neg_inf = jnp.float32(-jnp.inf)

        def step(i, carry):
            l1v = l1[...]
            m = jnp.max(l1v)
            q = jnp.min(jnp.where(l1v == m, riota, big))
            t = q // 128
            s = q - t * 128
            row = f3[pl.ds(t, 1), pl.ds(s, 1), :]
            lane = jnp.min(jnp.where(row == m, liota, big))
            newrow = jnp.where(liota == lane, neg_inf, row)
            f3[pl.ds(t, 1), pl.ds(s, 1), :] = newrow
            l1row = l1[pl.ds(t, 1), :]
            l1[pl.ds(t, 1), :] = jnp.where(siota == s,
                                           jnp.max(newrow, axis=2), l1row)

            sc_o[:, pl.ds(i, 1), :] = jnp.reshape(jax.nn.sigmoid(m),
                                                  (1, 1, 1))
            lb_o[:, pl.ds(i, 1), :] = jnp.reshape(lane, (1, 1, 1))

            row68 = cb[:, pl.ds(q, 1), :]
            cd_o[:, pl.ds(i, 1), :] = row68[:, :, 0:2 * _P] * svec
            cx = row68[0, 0, 2 * _P + 0]
            cy = row68[0, 0, 2 * _P + 1]
            w = row68[0, 0, 2 * _P + 2]
            h = row68[0, 0, 2 * _P + 3]
            x0 = (cx - 0.5 * w) * o0
            y0 = (cy - 0.5 * h) * o1
            x1 = (cx + 0.5 * w) * o0
            y1 = (cy + 0.5 * h) * o1
            bo_o[:, pl.ds(i, 1), :] = jnp.where(
                biota == 0, x0, jnp.where(biota == 1, y0,
                                          jnp.where(biota == 2, x1, y1)))
            return carry

        jax.lax.fori_loop(0, _TOPK, step, 0)


@functools.partial(jax.jit, static_argnames=("interpret",))
def _run(pred_logits, pred_coords, pred_boxes, orig_target_sizes, interpret=False):
    comb = jnp.concatenate(
        [pred_coords.reshape(_B, _N, 2 * _P), pred_boxes], axis=2)
    ots = orig_target_sizes.reshape(_B, 1, 2)

    grid = (_B, _NBLK)
    lb, bo, cd, sc = pl.pallas_call(
        _body,
        grid=grid,
        in_specs=[
            pl.BlockSpec((1, _QB, _C), lambda b, a: (b, a, 0)),
            pl.BlockSpec((1, _N, _D), lambda b, a: (b, 0, 0)),
            pl.BlockSpec((1, 1, 2), lambda b, a: (b, 0, 0)),
        ],
        out_specs=[
            pl.BlockSpec((1, _TOPK, 1), lambda b, a: (b, 0, 0)),
            pl.BlockSpec((1, _TOPK, 4), lambda b, a: (b, 0, 0)),
            pl.BlockSpec((1, _TOPK, 2 * _P), lambda b, a: (b, 0, 0)),
            pl.BlockSpec((1, _TOPK, 1), lambda b, a: (b, 0, 0)),
        ],
        out_shape=[
            jax.ShapeDtypeStruct((_B, _TOPK, 1), jnp.int32),
            jax.ShapeDtypeStruct((_B, _TOPK, 4), jnp.float32),
            jax.ShapeDtypeStruct((_B, _TOPK, 2 * _P), jnp.float32),
            jax.ShapeDtypeStruct((_B, _TOPK, 1), jnp.float32),
        ],
        scratch_shapes=[
            pltpu.VMEM((_T, 128, _C), jnp.float32),
            pltpu.VMEM((_T, 128), jnp.float32),
        ],
        interpret=interpret,
    )(pred_logits, comb, ots)

    labels = lb.reshape(_B, _TOPK)
    boxes_sel = bo
    coords_sel = cd.reshape(_B, _TOPK, _P, 2)
    top_scores = sc.reshape(_B, _TOPK)
    return labels, boxes_sel, coords_sel, top_scores


def kernel(pred_logits, pred_coords, pred_boxes, orig_target_sizes, input_sizes):
    return _run(pred_logits, pred_coords, pred_boxes, orig_target_sizes)
